# baseline (speedup 1.0000x reference)
import functools

import jax
import jax.numpy as jnp
from jax import lax
from jax.experimental import pallas as pl
from jax.experimental.pallas import tpu as pltpu


def _router_body(x_ref, wr_ref, comb_ref):
    xv = x_ref[...]
    logits = lax.dot_general(
        xv, wr_ref[...], (((1,), (1,)), ((), ())),
        preferred_element_type=jnp.float32)
    m = jnp.max(logits, axis=-1, keepdims=True)
    p = jnp.exp(logits - m)
    p = p / jnp.sum(p, axis=-1, keepdims=True)
    T, E = p.shape
    idxs = lax.broadcasted_iota(jnp.int32, (T, E), 1)
    m1 = jnp.max(p, axis=-1, keepdims=True)
    i1 = jnp.min(jnp.where(p == m1, idxs, E), axis=-1, keepdims=True)
    sel1 = idxs == i1
    p2 = jnp.where(sel1, -jnp.inf, p)
    m2 = jnp.max(p2, axis=-1, keepdims=True)
    i2 = jnp.min(jnp.where(p2 == m2, idxs, E), axis=-1, keepdims=True)
    sel2 = idxs == i2
    denom = m1 + m2
    comb_ref[...] = (jnp.where(sel1, m1 / denom, 0.0)
                     + jnp.where(sel2, m2 / denom, 0.0))


def _router(x_flat, Wr):
    T, H = x_flat.shape
    E = Wr.shape[0]
    return pl.pallas_call(
        _router_body,
        out_shape=jax.ShapeDtypeStruct((T, E), jnp.float32),
    )(x_flat, Wr)


def _fc1_body(te_ref, xs_ref, w1g_ref, w1u_ref, sw_ref, act_ref):
    xv = xs_ref[...].astype(jnp.bfloat16)
    gate = lax.dot_general(
        xv, w1g_ref[0].astype(jnp.bfloat16), (((1,), (1,)), ((), ())),
        preferred_element_type=jnp.float32)
    up = lax.dot_general(
        xv, w1u_ref[0].astype(jnp.bfloat16), (((1,), (1,)), ((), ())),
        preferred_element_type=jnp.float32)
    a = gate * jax.nn.sigmoid(gate) * up
    act_ref[...] = (a * sw_ref[...]).astype(jnp.bfloat16)


def _fc1(te, xs, w1g, w1u, sw, tt, it):
    P, H = xs.shape
    E, I, _ = w1g.shape
    npt, ni = P // tt, I // it
    grid_spec = pltpu.PrefetchScalarGridSpec(
        num_scalar_prefetch=1,
        grid=(ni, npt),
        in_specs=[
            pl.BlockSpec((tt, H), lambda i, p, te_r: (p, 0)),
            pl.BlockSpec((1, it, H), lambda i, p, te_r: (te_r[p], i, 0)),
            pl.BlockSpec((1, it, H), lambda i, p, te_r: (te_r[p], i, 0)),
            pl.BlockSpec((tt, 1), lambda i, p, te_r: (p, 0)),
        ],
        out_specs=pl.BlockSpec((tt, it), lambda i, p, te_r: (p, i)),
    )
    return pl.pallas_call(
        _fc1_body,
        grid_spec=grid_spec,
        out_shape=jax.ShapeDtypeStruct((P, I), jnp.bfloat16),
        compiler_params=pltpu.CompilerParams(
            dimension_semantics=("arbitrary", "arbitrary"),
        ),
    )(te, xs, w1g, w1u, sw)


def _fc2_body(te_ref, act_ref, w2_ref, ys_ref):
    ys_ref[...] = lax.dot_general(
        act_ref[...], w2_ref[0].astype(jnp.bfloat16),
        (((1,), (1,)), ((), ())),
        preferred_element_type=jnp.float32)


def _fc2(te, act, w2, tt, th):
    P, I = act.shape
    E, H, _ = w2.shape
    npt, nh = P // tt, H // th
    grid_spec = pltpu.PrefetchScalarGridSpec(
        num_scalar_prefetch=1,
        grid=(nh, npt),
        in_specs=[
            pl.BlockSpec((tt, I), lambda h, p, te_r: (p, 0)),
            pl.BlockSpec((1, th, I), lambda h, p, te_r: (te_r[p], h, 0)),
        ],
        out_specs=pl.BlockSpec((tt, th), lambda h, p, te_r: (p, h)),
    )
    return pl.pallas_call(
        _fc2_body,
        grid_spec=grid_spec,
        out_shape=jax.ShapeDtypeStruct((P, H), jnp.float32),
        compiler_params=pltpu.CompilerParams(
            dimension_semantics=("arbitrary", "arbitrary"),
        ),
    )(te, act, w2)


def _add_body(a_ref, b_ref, o_ref):
    o_ref[...] = a_ref[...] + b_ref[...]


def _combine_add(ya, yb, tblk):
    T, H = ya.shape
    return pl.pallas_call(
        _add_body,
        grid=(T // tblk,),
        in_specs=[
            pl.BlockSpec((tblk, H), lambda i: (i, 0)),
            pl.BlockSpec((tblk, H), lambda i: (i, 0)),
        ],
        out_specs=pl.BlockSpec((tblk, H), lambda i: (i, 0)),
        out_shape=jax.ShapeDtypeStruct((T, H), jnp.float32),
    )(ya, yb)


@functools.partial(jax.jit, static_argnames=("tt", "it", "th"))
def _moe(x_flat, Wr, w1g, w1u, w2, tt=128, it=512, th=512):
    T, H = x_flat.shape
    E = Wr.shape[0]
    P = 2 * T + E * tt
    npt = P // tt

    combine = _router(x_flat, Wr)

    ints = jnp.int32
    e1 = jnp.argmax(combine, axis=1).astype(ints)
    oh1 = lax.broadcasted_iota(ints, (T, E), 1) == e1[:, None]
    e2 = jnp.argmax(jnp.where(oh1, -1.0, combine), axis=1).astype(ints)
    ef = jnp.stack([e1, e2], axis=1).reshape(-1)
    order = jnp.argsort(ef).astype(ints)
    ef_s = ef[order]
    tok_s = (order // 2).astype(ints)
    counts = jnp.sum((ef[:, None] == jnp.arange(E)[None, :]), axis=0,
                     dtype=ints)
    gstart = jnp.concatenate([jnp.zeros(1, ints), jnp.cumsum(counts)[:-1]])
    pc = ((counts + tt - 1) // tt) * tt
    cpc = jnp.cumsum(pc)
    pstart = jnp.concatenate([jnp.zeros(1, ints), cpc[:-1]])
    rank = jnp.arange(2 * T, dtype=ints) - gstart[ef_s]
    slot = (pstart[ef_s] + rank).astype(ints)
    perm_tok = jnp.zeros(P, ints).at[slot].set(tok_s)
    sw = jnp.zeros(P, jnp.float32).at[slot].set(combine[tok_s, ef_s])
    tile_expert = jnp.clip(
        jnp.searchsorted(cpc, jnp.arange(npt, dtype=ints) * tt, side="right"),
        0, E - 1).astype(ints)
    inv = jnp.zeros(2 * T, ints).at[order].set(slot)
    s1, s2 = inv[0::2], inv[1::2]

    xs = jnp.take(x_flat, perm_tok, axis=0)

    act = _fc1(tile_expert, xs, w1g, w1u, sw[:, None], tt, it)
    ys = _fc2(tile_expert, act, w2, tt, th)

    ya = jnp.take(ys, s1, axis=0)
    yb = jnp.take(ys, s2, axis=0)
    return _combine_add(ya, yb, min(256, T))


def kernel(x, Wr, w1, w2):
    b, s, h = x.shape
    x_flat = x.reshape(-1, h)
    I = w1.shape[1] // 2
    w1g = w1[:, :I, :]
    w1u = w1[:, I:, :]
    out = _moe(x_flat, Wr, w1g, w1u, w2)
    return out.reshape(b, s, h)


# speedup vs baseline: 1.0496x; 1.0496x over previous
import functools

import jax
import jax.numpy as jnp
from jax import lax
from jax.experimental import pallas as pl
from jax.experimental.pallas import tpu as pltpu
from jax.experimental.pallas import tpu_sc as plsc


_SC_NC, _SC_NS = 2, 16
_SC_NW = _SC_NC * _SC_NS
_SC_CH = 16


@functools.lru_cache(maxsize=None)
def _sc_gather_builder(V, P, H):
    bpw = P // _SC_NW
    nch = bpw // _SC_CH
    mesh = plsc.VectorSubcoreMesh(
        core_axis_name="c", subcore_axis_name="s", num_cores=_SC_NC)

    @functools.partial(
        pl.kernel, mesh=mesh,
        out_type=jax.ShapeDtypeStruct((P, H), jnp.float32),
        scratch_types=[
            pltpu.VMEM((bpw,), jnp.int32),
            pltpu.VMEM((_SC_CH, H), jnp.float32),
            pltpu.SemaphoreType.DMA,
        ],
    )
    def k(table_hbm, idx_hbm, out_hbm, idx_v, rows_v, sem):
        wid = lax.axis_index("s") * _SC_NC + lax.axis_index("c")
        base = wid * bpw
        pltpu.sync_copy(idx_hbm.at[pl.ds(base, bpw)], idx_v)

        def body(c, carry):
            off = c * _SC_CH
            pltpu.async_copy(
                table_hbm.at[idx_v.at[pl.ds(off, _SC_CH)]], rows_v, sem
            ).wait()
            pltpu.sync_copy(rows_v, out_hbm.at[pl.ds(base + off, _SC_CH)])
            return carry

        lax.fori_loop(0, nch, body, 0)

    return k


def _sc_gather(table, idx):
    V, H = table.shape
    (P,) = idx.shape
    return _sc_gather_builder(V, P, H)(table, idx)


def _router_body(x_ref, wr_ref, comb_ref):
    xv = x_ref[...]
    logits = lax.dot_general(
        xv, wr_ref[...], (((1,), (1,)), ((), ())),
        preferred_element_type=jnp.float32)
    m = jnp.max(logits, axis=-1, keepdims=True)
    p = jnp.exp(logits - m)
    p = p / jnp.sum(p, axis=-1, keepdims=True)
    T, E = p.shape
    idxs = lax.broadcasted_iota(jnp.int32, (T, E), 1)
    m1 = jnp.max(p, axis=-1, keepdims=True)
    i1 = jnp.min(jnp.where(p == m1, idxs, E), axis=-1, keepdims=True)
    sel1 = idxs == i1
    p2 = jnp.where(sel1, -jnp.inf, p)
    m2 = jnp.max(p2, axis=-1, keepdims=True)
    i2 = jnp.min(jnp.where(p2 == m2, idxs, E), axis=-1, keepdims=True)
    sel2 = idxs == i2
    denom = m1 + m2
    comb_ref[...] = (jnp.where(sel1, m1 / denom, 0.0)
                     + jnp.where(sel2, m2 / denom, 0.0))


def _router(x_flat, Wr):
    T, H = x_flat.shape
    E = Wr.shape[0]
    return pl.pallas_call(
        _router_body,
        out_shape=jax.ShapeDtypeStruct((T, E), jnp.float32),
    )(x_flat, Wr)


def _fc1_body(te_ref, xs_ref, w1g_ref, w1u_ref, sw_ref, act_ref):
    xv = xs_ref[...].astype(jnp.bfloat16)
    gate = lax.dot_general(
        xv, w1g_ref[0].astype(jnp.bfloat16), (((1,), (1,)), ((), ())),
        preferred_element_type=jnp.float32)
    up = lax.dot_general(
        xv, w1u_ref[0].astype(jnp.bfloat16), (((1,), (1,)), ((), ())),
        preferred_element_type=jnp.float32)
    a = gate * jax.nn.sigmoid(gate) * up
    act_ref[...] = (a * sw_ref[...]).astype(jnp.bfloat16)


def _fc1(te, xs, w1g, w1u, sw, tt, it):
    P, H = xs.shape
    E, I, _ = w1g.shape
    npt, ni = P // tt, I // it
    grid_spec = pltpu.PrefetchScalarGridSpec(
        num_scalar_prefetch=1,
        grid=(ni, npt),
        in_specs=[
            pl.BlockSpec((tt, H), lambda i, p, te_r: (p, 0)),
            pl.BlockSpec((1, it, H), lambda i, p, te_r: (te_r[p], i, 0)),
            pl.BlockSpec((1, it, H), lambda i, p, te_r: (te_r[p], i, 0)),
            pl.BlockSpec((tt, 1), lambda i, p, te_r: (p, 0)),
        ],
        out_specs=pl.BlockSpec((tt, it), lambda i, p, te_r: (p, i)),
    )
    return pl.pallas_call(
        _fc1_body,
        grid_spec=grid_spec,
        out_shape=jax.ShapeDtypeStruct((P, I), jnp.bfloat16),
        compiler_params=pltpu.CompilerParams(
            dimension_semantics=("arbitrary", "arbitrary"),
        ),
    )(te, xs, w1g, w1u, sw)


def _fc2_body(te_ref, act_ref, w2_ref, ys_ref):
    ys_ref[...] = lax.dot_general(
        act_ref[...], w2_ref[0].astype(jnp.bfloat16),
        (((1,), (1,)), ((), ())),
        preferred_element_type=jnp.float32)


def _fc2(te, act, w2, tt, th):
    P, I = act.shape
    E, H, _ = w2.shape
    npt, nh = P // tt, H // th
    grid_spec = pltpu.PrefetchScalarGridSpec(
        num_scalar_prefetch=1,
        grid=(nh, npt),
        in_specs=[
            pl.BlockSpec((tt, I), lambda h, p, te_r: (p, 0)),
            pl.BlockSpec((1, th, I), lambda h, p, te_r: (te_r[p], h, 0)),
        ],
        out_specs=pl.BlockSpec((tt, th), lambda h, p, te_r: (p, h)),
    )
    return pl.pallas_call(
        _fc2_body,
        grid_spec=grid_spec,
        out_shape=jax.ShapeDtypeStruct((P, H), jnp.float32),
        compiler_params=pltpu.CompilerParams(
            dimension_semantics=("arbitrary", "arbitrary"),
        ),
    )(te, act, w2)


def _add_body(a_ref, b_ref, o_ref):
    o_ref[...] = a_ref[...] + b_ref[...]


def _combine_add(ya, yb, tblk):
    T, H = ya.shape
    return pl.pallas_call(
        _add_body,
        grid=(T // tblk,),
        in_specs=[
            pl.BlockSpec((tblk, H), lambda i: (i, 0)),
            pl.BlockSpec((tblk, H), lambda i: (i, 0)),
        ],
        out_specs=pl.BlockSpec((tblk, H), lambda i: (i, 0)),
        out_shape=jax.ShapeDtypeStruct((T, H), jnp.float32),
    )(ya, yb)


@functools.partial(jax.jit, static_argnames=("tt", "it", "th"))
def _moe(x_flat, Wr, w1g, w1u, w2, tt=128, it=512, th=512):
    T, H = x_flat.shape
    E = Wr.shape[0]
    P = 2 * T + E * tt
    npt = P // tt

    combine = _router(x_flat, Wr)

    ints = jnp.int32
    e1 = jnp.argmax(combine, axis=1).astype(ints)
    oh1 = lax.broadcasted_iota(ints, (T, E), 1) == e1[:, None]
    e2 = jnp.argmax(jnp.where(oh1, -1.0, combine), axis=1).astype(ints)
    ef = jnp.stack([e1, e2], axis=1).reshape(-1)
    oh = (ef[:, None] == jnp.arange(E, dtype=ints)[None, :]).astype(ints)
    csum = jnp.cumsum(oh, axis=0)
    rank = jnp.sum((csum - oh) * oh, axis=1)
    counts = csum[-1]
    pc = ((counts + tt - 1) // tt) * tt
    cpc = jnp.cumsum(pc)
    pstart = jnp.concatenate([jnp.zeros(1, ints), cpc[:-1]])
    slot = (pstart[ef] + rank).astype(ints)
    tok = (jnp.arange(2 * T, dtype=ints) // 2)
    perm_tok = jnp.zeros(P, ints).at[slot].set(tok, unique_indices=True)
    sw = jnp.zeros(P, jnp.float32).at[slot].set(
        jnp.take_along_axis(combine, ef.reshape(T, 2), axis=1).reshape(-1),
        unique_indices=True)
    starts = jnp.arange(npt, dtype=ints) * tt
    tile_expert = jnp.clip(
        jnp.sum(starts[:, None] >= cpc[None, :], axis=1), 0, E - 1
    ).astype(ints)
    s1, s2 = slot[0::2], slot[1::2]

    xs = _sc_gather(x_flat, perm_tok)

    act = _fc1(tile_expert, xs, w1g, w1u, sw[:, None], tt, it)
    ys = _fc2(tile_expert, act, w2, tt, th)

    ya = _sc_gather(ys, s1)
    yb = _sc_gather(ys, s2)
    return _combine_add(ya, yb, min(256, T))


def kernel(x, Wr, w1, w2):
    b, s, h = x.shape
    x_flat = x.reshape(-1, h)
    I = w1.shape[1] // 2
    w1g = w1[:, :I, :]
    w1u = w1[:, I:, :]
    out = _moe(x_flat, Wr, w1g, w1u, w2)
    return out.reshape(b, s, h)


# speedup vs baseline: 1.1801x; 1.1243x over previous
import functools

import jax
import jax.numpy as jnp
from jax import lax
from jax.experimental import pallas as pl
from jax.experimental.pallas import tpu as pltpu
from jax.experimental.pallas import tpu_sc as plsc


_SC_NC, _SC_NS = 2, 16
_SC_NW = _SC_NC * _SC_NS
_SC_CH = 32


@functools.lru_cache(maxsize=None)
def _sc_gather_builder(V, P, H):
    bpw = P // _SC_NW
    nch = bpw // _SC_CH
    mesh = plsc.VectorSubcoreMesh(
        core_axis_name="c", subcore_axis_name="s", num_cores=_SC_NC)

    @functools.partial(
        pl.kernel, mesh=mesh,
        out_type=jax.ShapeDtypeStruct((P, H), jnp.float32),
        scratch_types=[
            pltpu.VMEM((bpw,), jnp.int32),
            pltpu.VMEM((_SC_CH, H), jnp.float32),
            pltpu.SemaphoreType.DMA,
        ],
    )
    def k(table_hbm, idx_hbm, out_hbm, idx_v, rows_v, sem):
        wid = lax.axis_index("s") * _SC_NC + lax.axis_index("c")
        base = wid * bpw
        pltpu.sync_copy(idx_hbm.at[pl.ds(base, bpw)], idx_v)

        def body(c, carry):
            off = c * _SC_CH
            pltpu.async_copy(
                table_hbm.at[idx_v.at[pl.ds(off, _SC_CH)]], rows_v, sem
            ).wait()
            pltpu.sync_copy(rows_v, out_hbm.at[pl.ds(base + off, _SC_CH)])
            return carry

        lax.fori_loop(0, nch, body, 0)

    return k


def _sc_gather(table, idx):
    V, H = table.shape
    (P,) = idx.shape
    return _sc_gather_builder(V, P, H)(table, idx)


def _router_body(x_ref, wr_ref, comb_ref):
    xv = x_ref[...]
    logits = lax.dot_general(
        xv, wr_ref[...], (((1,), (1,)), ((), ())),
        preferred_element_type=jnp.float32)
    m = jnp.max(logits, axis=-1, keepdims=True)
    p = jnp.exp(logits - m)
    p = p / jnp.sum(p, axis=-1, keepdims=True)
    T, E = p.shape
    idxs = lax.broadcasted_iota(jnp.int32, (T, E), 1)
    m1 = jnp.max(p, axis=-1, keepdims=True)
    i1 = jnp.min(jnp.where(p == m1, idxs, E), axis=-1, keepdims=True)
    sel1 = idxs == i1
    p2 = jnp.where(sel1, -jnp.inf, p)
    m2 = jnp.max(p2, axis=-1, keepdims=True)
    i2 = jnp.min(jnp.where(p2 == m2, idxs, E), axis=-1, keepdims=True)
    sel2 = idxs == i2
    denom = m1 + m2
    comb_ref[...] = (jnp.where(sel1, m1 / denom, 0.0)
                     + jnp.where(sel2, m2 / denom, 0.0))


def _router(x_flat, Wr):
    T, H = x_flat.shape
    E = Wr.shape[0]
    return pl.pallas_call(
        _router_body,
        out_shape=jax.ShapeDtypeStruct((T, E), jnp.float32),
    )(x_flat, Wr)


def _fc1_body(te_ref, xs_ref, w1g_ref, w1u_ref, sw_ref, act_ref):
    xv = xs_ref[...].astype(jnp.bfloat16)
    gate = lax.dot_general(
        xv, w1g_ref[0].astype(jnp.bfloat16), (((1,), (1,)), ((), ())),
        preferred_element_type=jnp.float32)
    up = lax.dot_general(
        xv, w1u_ref[0].astype(jnp.bfloat16), (((1,), (1,)), ((), ())),
        preferred_element_type=jnp.float32)
    a = gate * jax.nn.sigmoid(gate) * up
    act_ref[...] = (a * sw_ref[...]).astype(jnp.bfloat16)


def _fc1(te, xs, w1g, w1u, sw, tt, it):
    P, H = xs.shape
    E, I, _ = w1g.shape
    npt, ni = P // tt, I // it
    grid_spec = pltpu.PrefetchScalarGridSpec(
        num_scalar_prefetch=1,
        grid=(ni, npt),
        in_specs=[
            pl.BlockSpec((tt, H), lambda i, p, te_r: (p, 0)),
            pl.BlockSpec((1, it, H), lambda i, p, te_r: (te_r[p], i, 0)),
            pl.BlockSpec((1, it, H), lambda i, p, te_r: (te_r[p], i, 0)),
            pl.BlockSpec((tt, 1), lambda i, p, te_r: (p, 0)),
        ],
        out_specs=pl.BlockSpec((tt, it), lambda i, p, te_r: (p, i)),
    )
    return pl.pallas_call(
        _fc1_body,
        grid_spec=grid_spec,
        out_shape=jax.ShapeDtypeStruct((P, I), jnp.bfloat16),
        compiler_params=pltpu.CompilerParams(
            dimension_semantics=("arbitrary", "arbitrary"),
        ),
    )(te, xs, w1g, w1u, sw)


def _fc2_body(te_ref, act_ref, w2_ref, ys_ref):
    ys_ref[...] = lax.dot_general(
        act_ref[...], w2_ref[0].astype(jnp.bfloat16),
        (((1,), (1,)), ((), ())),
        preferred_element_type=jnp.float32)


def _fc2(te, act, w2, tt, th):
    P, I = act.shape
    E, H, _ = w2.shape
    npt, nh = P // tt, H // th
    grid_spec = pltpu.PrefetchScalarGridSpec(
        num_scalar_prefetch=1,
        grid=(nh, npt),
        in_specs=[
            pl.BlockSpec((tt, I), lambda h, p, te_r: (p, 0)),
            pl.BlockSpec((1, th, I), lambda h, p, te_r: (te_r[p], h, 0)),
        ],
        out_specs=pl.BlockSpec((tt, th), lambda h, p, te_r: (p, h)),
    )
    return pl.pallas_call(
        _fc2_body,
        grid_spec=grid_spec,
        out_shape=jax.ShapeDtypeStruct((P, H), jnp.float32),
        compiler_params=pltpu.CompilerParams(
            dimension_semantics=("arbitrary", "arbitrary"),
        ),
    )(te, act, w2)


def _add_body(a_ref, b_ref, o_ref):
    o_ref[...] = a_ref[...] + b_ref[...]


def _combine_add(ya, yb, tblk):
    T, H = ya.shape
    return pl.pallas_call(
        _add_body,
        grid=(T // tblk,),
        in_specs=[
            pl.BlockSpec((tblk, H), lambda i: (i, 0)),
            pl.BlockSpec((tblk, H), lambda i: (i, 0)),
        ],
        out_specs=pl.BlockSpec((tblk, H), lambda i: (i, 0)),
        out_shape=jax.ShapeDtypeStruct((T, H), jnp.float32),
    )(ya, yb)


@functools.partial(jax.jit, static_argnames=("tt", "it", "th"))
def _moe(x_flat, Wr, w1g, w1u, w2, tt=128, it=1024, th=2048):
    T, H = x_flat.shape
    E = Wr.shape[0]
    P = 2 * T + E * tt
    npt = P // tt

    combine = _router(x_flat, Wr)

    ints = jnp.int32
    e1 = jnp.argmax(combine, axis=1).astype(ints)
    oh1 = lax.broadcasted_iota(ints, (T, E), 1) == e1[:, None]
    e2 = jnp.argmax(jnp.where(oh1, -1.0, combine), axis=1).astype(ints)
    ef = jnp.stack([e1, e2], axis=1).reshape(-1)
    oh = (ef[:, None] == jnp.arange(E, dtype=ints)[None, :])
    ohf = oh.astype(jnp.float32).reshape(-1, 128, E)
    ltri = (lax.broadcasted_iota(ints, (128, 128), 0)
            >= lax.broadcasted_iota(ints, (128, 128), 1)).astype(jnp.float32)
    within = jnp.einsum("rc,kce->kre", ltri, ohf,
                        precision=lax.Precision.HIGHEST)
    chunk_tot = ohf.sum(axis=1)
    C = chunk_tot.shape[0]
    excl_chunk = (jnp.cumsum(chunk_tot, axis=0) - chunk_tot)
    csum = (within + excl_chunk[:, None, :]).reshape(2 * T, E)
    rank = jnp.sum((csum - ohf.reshape(2 * T, E)) * ohf.reshape(2 * T, E),
                   axis=1).astype(ints)
    counts = csum[-1].astype(ints)
    pc = ((counts + tt - 1) // tt) * tt
    cpc = jnp.cumsum(pc)
    pstart = jnp.concatenate([jnp.zeros(1, ints), cpc[:-1]])
    slot = (pstart[ef] + rank).astype(ints)
    tok = (jnp.arange(2 * T, dtype=ints) // 2)
    perm_tok = jnp.zeros(P, ints).at[slot].set(tok, unique_indices=True)
    sw = jnp.zeros(P, jnp.float32).at[slot].set(
        jnp.take_along_axis(combine, ef.reshape(T, 2), axis=1).reshape(-1),
        unique_indices=True)
    starts = jnp.arange(npt, dtype=ints) * tt
    tile_expert = jnp.clip(
        jnp.sum(starts[:, None] >= cpc[None, :], axis=1), 0, E - 1
    ).astype(ints)
    s1, s2 = slot[0::2], slot[1::2]

    xs = _sc_gather(x_flat, perm_tok)

    act = _fc1(tile_expert, xs, w1g, w1u, sw[:, None], tt, it)
    ys = _fc2(tile_expert, act, w2, tt, th)

    ya = _sc_gather(ys, s1)
    yb = _sc_gather(ys, s2)
    return _combine_add(ya, yb, min(256, T))


def kernel(x, Wr, w1, w2):
    b, s, h = x.shape
    x_flat = x.reshape(-1, h)
    I = w1.shape[1] // 2
    w1g = w1[:, :I, :]
    w1u = w1[:, I:, :]
    out = _moe(x_flat, Wr, w1g, w1u, w2)
    return out.reshape(b, s, h)


# speedup vs baseline: 2.1187x; 1.7954x over previous
import functools

import jax
import jax.numpy as jnp
from jax.experimental import pallas as pl
from jax.experimental.pallas import tpu as pltpu


def _moe_body(x_ref, wr_ref, w1g_ref, w1u_ref, w2_ref, out_ref, comb_ref):
    e = pl.program_id(0)
    i = pl.program_id(1)

    @pl.when((e == 0) & (i == 0))
    def _router():
        xv = x_ref[...]
        logits = jax.lax.dot_general(
            xv, wr_ref[...], (((1,), (1,)), ((), ())),
            preferred_element_type=jnp.float32)
        m = jnp.max(logits, axis=-1, keepdims=True)
        p = jnp.exp(logits - m)
        p = p / jnp.sum(p, axis=-1, keepdims=True)
        T, E = p.shape
        idxs = jax.lax.broadcasted_iota(jnp.int32, (T, E), 1)
        m1 = jnp.max(p, axis=-1, keepdims=True)
        i1 = jnp.min(jnp.where(p == m1, idxs, E), axis=-1, keepdims=True)
        sel1 = idxs == i1
        p2 = jnp.where(sel1, -jnp.inf, p)
        m2 = jnp.max(p2, axis=-1, keepdims=True)
        i2 = jnp.min(jnp.where(p2 == m2, idxs, E), axis=-1, keepdims=True)
        sel2 = idxs == i2
        denom = m1 + m2
        comb_ref[...] = (jnp.where(sel1, m1 / denom, 0.0)
                         + jnp.where(sel2, m2 / denom, 0.0))
        out_ref[...] = jnp.zeros_like(out_ref)

    xv = x_ref[...].astype(jnp.bfloat16)
    gate = jax.lax.dot_general(
        xv, w1g_ref[0].astype(jnp.bfloat16), (((1,), (1,)), ((), ())),
        preferred_element_type=jnp.float32)
    up = jax.lax.dot_general(
        xv, w1u_ref[0].astype(jnp.bfloat16), (((1,), (1,)), ((), ())),
        preferred_element_type=jnp.float32)
    act = gate * jax.nn.sigmoid(gate) * up
    comb = comb_ref[...]
    lane = jax.lax.broadcasted_iota(jnp.int32, comb.shape, 1)
    cw = jnp.sum(jnp.where(lane == e, comb, 0.0), axis=1, keepdims=True)
    act = (act * cw).astype(jnp.bfloat16)
    out_ref[...] += jax.lax.dot_general(
        act, w2_ref[0].astype(jnp.bfloat16), (((1,), (1,)), ((), ())),
        preferred_element_type=jnp.float32)


@functools.partial(jax.jit, static_argnames=("it",))
def _moe(x_flat, Wr, w1, w2, it=256):
    T, H = x_flat.shape
    E = Wr.shape[0]
    I = w1.shape[1] // 2
    ni = I // it
    out = pl.pallas_call(
        _moe_body,
        grid=(E, ni),
        in_specs=[
            pl.BlockSpec((T, H), lambda e, i: (0, 0)),
            pl.BlockSpec((E, H), lambda e, i: (0, 0)),
            pl.BlockSpec((1, it, H), lambda e, i: (e, i, 0)),
            pl.BlockSpec((1, it, H), lambda e, i: (e, I // it + i, 0)),
            pl.BlockSpec((1, H, it), lambda e, i: (e, 0, i)),
        ],
        out_specs=pl.BlockSpec((T, H), lambda e, i: (0, 0)),
        out_shape=jax.ShapeDtypeStruct((T, H), jnp.float32),
        scratch_shapes=[pltpu.VMEM((T, E), jnp.float32)],
        compiler_params=pltpu.CompilerParams(
            dimension_semantics=("arbitrary", "arbitrary"),
        ),
    )(x_flat, Wr, w1, w1, w2)
    return out


def kernel(x, Wr, w1, w2):
    b, s, h = x.shape
    x_flat = x.reshape(-1, h)
    out = _moe(x_flat, Wr, w1, w2)
    return out.reshape(b, s, h)
